# Initial kernel scaffold; baseline (speedup 1.0000x reference)
import jax
import jax.numpy as jnp
from jax.experimental import pallas as pl


def kernel(logits):
    raise NotImplementedError("write your pallas kernel here")


# speedup vs baseline: 15.9167x; 15.9167x over previous
import functools

import jax
import jax.numpy as jnp
from jax.experimental import pallas as pl
from jax.experimental.pallas import tpu as pltpu

K = 64
TEMPERATURE = 1.0
EPS = 1e-10
BISECT_ITERS = 34


def _row_kernel(l_ref, g_ref, out_ref, scratch_ref):
    g = g_ref[0]
    lo0 = jnp.min(g) - 1.0
    hi0 = jnp.max(g)

    def body(_, carry):
        lo, hi = carry
        mid = 0.5 * (lo + hi)
        cnt = jnp.sum(jnp.where(g_ref[0] > mid, 1.0, 0.0))
        pred = cnt >= K
        return jnp.where(pred, mid, lo), jnp.where(pred, hi, mid)

    lo, hi = jax.lax.fori_loop(0, BISECT_ITERS, body, (lo0, hi0))
    t = 0.5 * (lo + hi)

    scratch_ref[...] = l_ref[0] * jax.nn.sigmoid((g_ref[0] - t) / TEMPERATURE)
    mx = jnp.max(scratch_ref[...])
    out_ref[0] = jnp.exp((scratch_ref[...] - mx) / TEMPERATURE)
    s = jnp.sum(out_ref[0])
    out_ref[0] = out_ref[0] * (1.0 / s)


@functools.partial(jax.jit, static_argnames=())
def _run(logits, g):
    n_rows, n_cols = logits.shape
    inner = 1000 if n_cols % 1000 == 0 else n_cols
    outer = n_cols // inner
    l3 = logits.reshape(n_rows, outer, inner)
    g3 = g.reshape(n_rows, outer, inner)
    out = pl.pallas_call(
        _row_kernel,
        grid=(n_rows,),
        in_specs=[
            pl.BlockSpec((1, outer, inner), lambda i: (i, 0, 0)),
            pl.BlockSpec((1, outer, inner), lambda i: (i, 0, 0)),
        ],
        out_specs=pl.BlockSpec((1, outer, inner), lambda i: (i, 0, 0)),
        out_shape=jax.ShapeDtypeStruct((n_rows, outer, inner), logits.dtype),
        scratch_shapes=[pltpu.VMEM((outer, inner), jnp.float32)],
    )(l3, g3)
    return out.reshape(n_rows, n_cols)


def kernel(logits):
    u = jax.random.uniform(jax.random.key(1), logits.shape, dtype=logits.dtype)
    gumbel_noise = -jnp.log(-jnp.log(u + EPS) + EPS)
    g = logits + gumbel_noise
    return _run(logits, g)
